# Initial kernel scaffold; baseline (speedup 1.0000x reference)
import functools

import jax
import jax.numpy as jnp
from jax import lax
from jax.experimental import pallas as pl
from jax.experimental.pallas import tpu as pltpu
from jax.experimental.pallas import tpu_sc as plsc

NV = 100000
NF = 800000
E = 1600000
B = 16
ITERS = 5
DAMP = 0.5

NV_PAD = 100096
ROWS = 6272
NF_PAD = ROWS * 128
NT = 32
ROWS_T = ROWS // NT
HROWS = ROWS_T // 2
VSLICE = NV_PAD // 16

BLK = 6272
GRID_F = NF_PAD // BLK
BLKV = 4352
GRID_V = NV_PAD // BLKV

_mesh = plsc.VectorSubcoreMesh(core_axis_name="c", subcore_axis_name="s")


def _zero_shared(zbuf, shared_refs, s):
  def body(i, _):
    zbuf[pl.ds(pl.multiple_of(i * 16, 16), 16)] = jnp.zeros((16,), jnp.float32)
    return 0
  lax.fori_loop(0, VSLICE // 16, body, 0)
  off = pl.multiple_of(s * VSLICE, 8)
  for ref in shared_refs:
    pltpu.sync_copy(zbuf, ref.at[pl.ds(off, VSLICE)])


@functools.partial(
    pl.kernel,
    out_type=jax.ShapeDtypeStruct((2, 2, NV_PAD), jnp.float32),
    mesh=_mesh,
    scratch_types=[
        pltpu.VMEM((HROWS, 128), jnp.int32),
        pltpu.VMEM((HROWS, 128), jnp.int32),
        pltpu.VMEM((HROWS, 128), jnp.float32),
        pltpu.VMEM((HROWS, 128), jnp.float32),
        pltpu.VMEM((HROWS, 128), jnp.float32),
        pltpu.VMEM((HROWS, 128), jnp.float32),
        pltpu.VMEM((VSLICE,), jnp.float32),
        pltpu.MemorySpace.VMEM_SHARED((NV_PAD,), jnp.float32),
        pltpu.MemorySpace.VMEM_SHARED((NV_PAD,), jnp.float32),
        pltpu.SemaphoreType.DMA,
    ],
)
def _sc_scatter(fvm_hbm, idx_hbm, out_hbm, i0, i1, v00, v01, v10, v11, zbuf,
                t0, t1, sem):
  c = lax.axis_index("c")
  s = lax.axis_index("s")
  wid = c * 16 + s
  _zero_shared(zbuf, (t0, t1), s)
  plsc.subcore_barrier()
  rbase = wid * ROWS_T
  for h in range(2):
    r0 = pl.multiple_of(rbase + h * HROWS, HROWS)
    loads = [
        pltpu.async_copy(idx_hbm.at[0, pl.ds(r0, HROWS), :], i0, sem),
        pltpu.async_copy(idx_hbm.at[1, pl.ds(r0, HROWS), :], i1, sem),
        pltpu.async_copy(fvm_hbm.at[0, pl.ds(r0, HROWS), :], v00, sem),
        pltpu.async_copy(fvm_hbm.at[1, pl.ds(r0, HROWS), :], v01, sem),
        pltpu.async_copy(fvm_hbm.at[2, pl.ds(r0, HROWS), :], v10, sem),
        pltpu.async_copy(fvm_hbm.at[3, pl.ds(r0, HROWS), :], v11, sem),
    ]
    for d in loads:
      d.wait()
    scats = [
        pltpu.async_copy(v00, t0.at[i0], sem, add=True),
        pltpu.async_copy(v01, t1.at[i0], sem, add=True),
        pltpu.async_copy(v10, t0.at[i1], sem, add=True),
        pltpu.async_copy(v11, t1.at[i1], sem, add=True),
    ]
    for d in scats:
      d.wait()
  plsc.subcore_barrier()
  off = pl.multiple_of(s * VSLICE, 8)
  pltpu.sync_copy(t0.at[pl.ds(off, VSLICE)],
                  out_hbm.at[c, 0, pl.ds(off, VSLICE)])
  pltpu.sync_copy(t1.at[pl.ds(off, VSLICE)],
                  out_hbm.at[c, 1, pl.ds(off, VSLICE)])


@functools.partial(
    pl.kernel,
    out_type=jax.ShapeDtypeStruct((2, NV_PAD), jnp.float32),
    mesh=_mesh,
    scratch_types=[
        pltpu.VMEM((HROWS, 128), jnp.int32),
        pltpu.VMEM((HROWS, 128), jnp.int32),
        pltpu.VMEM((HROWS, 128), jnp.float32),
        pltpu.VMEM((VSLICE,), jnp.float32),
        pltpu.MemorySpace.VMEM_SHARED((NV_PAD,), jnp.float32),
        pltpu.SemaphoreType.DMA,
    ],
)
def _sc_degrees(idx_hbm, ones_hbm, out_hbm, i0, i1, vones, zbuf, t0, sem):
  c = lax.axis_index("c")
  s = lax.axis_index("s")
  wid = c * 16 + s
  _zero_shared(zbuf, (t0,), s)
  pltpu.sync_copy(ones_hbm, vones)
  plsc.subcore_barrier()
  rbase = wid * ROWS_T
  for h in range(2):
    r0 = pl.multiple_of(rbase + h * HROWS, HROWS)
    loads = [
        pltpu.async_copy(idx_hbm.at[0, pl.ds(r0, HROWS), :], i0, sem),
        pltpu.async_copy(idx_hbm.at[1, pl.ds(r0, HROWS), :], i1, sem),
    ]
    for d in loads:
      d.wait()
    scats = [
        pltpu.async_copy(vones, t0.at[i0], sem, add=True),
        pltpu.async_copy(vones, t0.at[i1], sem, add=True),
    ]
    for d in scats:
      d.wait()
  plsc.subcore_barrier()
  off = pl.multiple_of(s * VSLICE, 8)
  pltpu.sync_copy(t0.at[pl.ds(off, VSLICE)], out_hbm.at[c, pl.ds(off, VSLICE)])


@functools.partial(
    pl.kernel,
    out_type=jax.ShapeDtypeStruct((4, ROWS, 128), jnp.float32),
    mesh=_mesh,
    scratch_types=[
        pltpu.VMEM((HROWS, 128), jnp.int32),
        pltpu.VMEM((HROWS, 128), jnp.int32),
        pltpu.VMEM((HROWS, 128), jnp.float32),
        pltpu.VMEM((HROWS, 128), jnp.float32),
        pltpu.VMEM((HROWS, 128), jnp.float32),
        pltpu.VMEM((HROWS, 128), jnp.float32),
        pltpu.MemorySpace.VMEM_SHARED((NV_PAD,), jnp.float32),
        pltpu.MemorySpace.VMEM_SHARED((NV_PAD,), jnp.float32),
        pltpu.SemaphoreType.DMA,
    ],
)
def _sc_gather(vb0_hbm, vb1_hbm, idx_hbm, out_hbm, i0, i1, g00, g01, g10, g11,
               t0, t1, sem):
  c = lax.axis_index("c")
  s = lax.axis_index("s")
  wid = c * 16 + s
  off = pl.multiple_of(s * VSLICE, 8)
  pltpu.sync_copy(vb0_hbm.at[pl.ds(off, VSLICE)], t0.at[pl.ds(off, VSLICE)])
  pltpu.sync_copy(vb1_hbm.at[pl.ds(off, VSLICE)], t1.at[pl.ds(off, VSLICE)])
  plsc.subcore_barrier()
  rbase = wid * ROWS_T
  for h in range(2):
    r0 = pl.multiple_of(rbase + h * HROWS, HROWS)
    loads = [
        pltpu.async_copy(idx_hbm.at[0, pl.ds(r0, HROWS), :], i0, sem),
        pltpu.async_copy(idx_hbm.at[1, pl.ds(r0, HROWS), :], i1, sem),
    ]
    for d in loads:
      d.wait()
    gats = [
        pltpu.async_copy(t0.at[i0], g00, sem),
        pltpu.async_copy(t1.at[i0], g01, sem),
        pltpu.async_copy(t0.at[i1], g10, sem),
        pltpu.async_copy(t1.at[i1], g11, sem),
    ]
    for d in gats:
      d.wait()
    stores = [
        pltpu.async_copy(g00, out_hbm.at[0, pl.ds(r0, HROWS), :], sem),
        pltpu.async_copy(g01, out_hbm.at[1, pl.ds(r0, HROWS), :], sem),
        pltpu.async_copy(g10, out_hbm.at[2, pl.ds(r0, HROWS), :], sem),
        pltpu.async_copy(g11, out_hbm.at[3, pl.ds(r0, HROWS), :], sem),
    ]
    for d in stores:
      d.wait()


def _lse2(a, b):
  m = jnp.maximum(a, b)
  return m + jnp.log1p(jnp.exp(-jnp.abs(a - b)))


def _seg16(fe, bids_col):
  k, n = fe.shape
  fe8 = jnp.concatenate([fe, jnp.zeros((8 - k, n), jnp.float32)], axis=0)
  oh = (bids_col == lax.broadcasted_iota(jnp.float32, (n, 128), 1))
  oh = oh.astype(jnp.float32)
  return lax.dot_general(fe8, oh, (((1,), (0,)), ((), ())),
                         precision=lax.Precision.HIGHEST,
                         preferred_element_type=jnp.float32)


def _dense_body(g_ref, vf_ref, fv_ref, po_ref, bf_ref, nvf_ref, nfv_ref,
                comp_ref):
  G = g_ref[...]
  V = vf_ref[...]
  F = fv_ref[...]
  P = po_ref[...]

  def r(x, p):
    return x[p:p + 1, :]

  nv = G - F
  l0 = _lse2(r(nv, 0), r(nv, 1))
  l1 = _lse2(r(nv, 2), r(nv, 3))
  lse_rows = jnp.concatenate([l0, l0, l1, l1], axis=0)
  Vn = DAMP * V + (1.0 - DAMP) * (nv - lse_rows)
  nvf_ref[...] = Vn
  m00 = _lse2(r(P, 0) + r(Vn, 2), r(P, 1) + r(Vn, 3))
  m01 = _lse2(r(P, 2) + r(Vn, 2), r(P, 3) + r(Vn, 3))
  m10 = _lse2(r(P, 0) + r(Vn, 0), r(P, 2) + r(Vn, 1))
  m11 = _lse2(r(P, 1) + r(Vn, 0), r(P, 3) + r(Vn, 1))
  z0 = _lse2(m00, m01)
  z1 = _lse2(m10, m11)
  nf = jnp.concatenate([m00 - z0, m01 - z0, m10 - z1, m11 - z1], axis=0)
  Fn = DAMP * F + (1.0 - DAMP) * nf
  nfv_ref[...] = Fn
  fb0 = r(P, 0) + r(Vn, 0) + r(Vn, 2)
  fb1 = r(P, 1) + r(Vn, 0) + r(Vn, 3)
  fb2 = r(P, 2) + r(Vn, 1) + r(Vn, 2)
  fb3 = r(P, 3) + r(Vn, 1) + r(Vn, 3)
  z = _lse2(_lse2(fb0, fb1), _lse2(fb2, fb3))
  FM = jnp.concatenate([fb0 - z, fb1 - z, fb2 - z, fb3 - z], axis=0)
  fe = jnp.exp(FM) * jnp.where(FM == -jnp.inf, 0.0, FM)
  part = _seg16(fe, bf_ref[...])

  @pl.when(pl.program_id(0) == 0)
  def _():
    comp_ref[...] = jnp.zeros_like(comp_ref)

  comp_ref[...] += part


def _tc_dense(g, vfm, fvm, pot, bf_col):
  return pl.pallas_call(
      _dense_body,
      grid=(GRID_F,),
      in_specs=[
          pl.BlockSpec((4, BLK), lambda i: (0, i)),
          pl.BlockSpec((4, BLK), lambda i: (0, i)),
          pl.BlockSpec((4, BLK), lambda i: (0, i)),
          pl.BlockSpec((4, BLK), lambda i: (0, i)),
          pl.BlockSpec((BLK, 1), lambda i: (i, 0)),
      ],
      out_specs=[
          pl.BlockSpec((4, BLK), lambda i: (0, i)),
          pl.BlockSpec((4, BLK), lambda i: (0, i)),
          pl.BlockSpec((8, 128), lambda i: (0, 0)),
      ],
      out_shape=[
          jax.ShapeDtypeStruct((4, NF_PAD), jnp.float32),
          jax.ShapeDtypeStruct((4, NF_PAD), jnp.float32),
          jax.ShapeDtypeStruct((8, 128), jnp.float32),
      ],
  )(g, vfm, fvm, pot, bf_col)


def _combine0_body(p_ref, dp_ref, vb0_ref, vb1_ref, deg_ref):
  Pp = p_ref[...]
  vb0_ref[...] = Pp[0, :] + Pp[2, :]
  vb1_ref[...] = Pp[1, :] + Pp[3, :]
  Dp = dp_ref[...]
  deg_ref[...] = Dp[0, :] + Dp[1, :]


def _tc_combine0(parts, deg_parts):
  return pl.pallas_call(
      _combine0_body,
      grid=(GRID_V,),
      in_specs=[
          pl.BlockSpec((4, BLKV), lambda i: (0, i)),
          pl.BlockSpec((2, BLKV), lambda i: (0, i)),
      ],
      out_specs=[
          pl.BlockSpec((BLKV,), lambda i: (i,)),
          pl.BlockSpec((BLKV,), lambda i: (i,)),
          pl.BlockSpec((BLKV,), lambda i: (i,)),
      ],
      out_shape=[
          jax.ShapeDtypeStruct((NV_PAD,), jnp.float32),
          jax.ShapeDtypeStruct((NV_PAD,), jnp.float32),
          jax.ShapeDtypeStruct((NV_PAD,), jnp.float32),
      ],
  )(parts, deg_parts)


def _combine_body(p_ref, deg_ref, bv_ref, vb0_ref, vb1_ref, comp_ref):
  Pp = p_ref[...]
  vb0 = Pp[0:1, :] + Pp[2:3, :]
  vb1 = Pp[1:2, :] + Pp[3:4, :]
  vb0_ref[...] = vb0[0, :]
  vb1_ref[...] = vb1[0, :]
  z = _lse2(vb0, vb1)
  vm0 = vb0 - z
  vm1 = vb1 - z
  w = deg_ref[...].reshape(1, -1) - 1.0
  fe0 = jnp.exp(vm0) * jnp.where(vm0 == -jnp.inf, 0.0, vm0) * w
  fe1 = jnp.exp(vm1) * jnp.where(vm1 == -jnp.inf, 0.0, vm1) * w
  part = _seg16(jnp.concatenate([fe0, fe1], axis=0), bv_ref[...])

  @pl.when(pl.program_id(0) == 0)
  def _():
    comp_ref[...] = jnp.zeros_like(comp_ref)

  comp_ref[...] += part


def _tc_combine(parts, deg, bv_col):
  return pl.pallas_call(
      _combine_body,
      grid=(GRID_V,),
      in_specs=[
          pl.BlockSpec((4, BLKV), lambda i: (0, i)),
          pl.BlockSpec((BLKV,), lambda i: (i,)),
          pl.BlockSpec((BLKV, 1), lambda i: (i, 0)),
      ],
      out_specs=[
          pl.BlockSpec((BLKV,), lambda i: (i,)),
          pl.BlockSpec((BLKV,), lambda i: (i,)),
          pl.BlockSpec((8, 128), lambda i: (0, 0)),
      ],
      out_shape=[
          jax.ShapeDtypeStruct((NV_PAD,), jnp.float32),
          jax.ShapeDtypeStruct((NV_PAD,), jnp.float32),
          jax.ShapeDtypeStruct((8, 128), jnp.float32),
      ],
  )(parts, deg, bv_col)


def _mlp_body(fi_ref, w1t_ref, b1_ref, w2t_ref, b2_ref, out_ref):
  h = lax.dot_general(fi_ref[...], w1t_ref[...], (((1,), (0,)), ((), ())),
                      precision=lax.Precision.HIGHEST,
                      preferred_element_type=jnp.float32) + b1_ref[...]
  out_ref[...] = lax.dot_general(h, w2t_ref[...], (((1,), (0,)), ((), ())),
                                 precision=lax.Precision.HIGHEST,
                                 preferred_element_type=jnp.float32
                                 ) + b2_ref[...]


def _tc_mlp(final_in, w1t, b1r, w2t, b2r):
  return pl.pallas_call(
      _mlp_body,
      out_shape=jax.ShapeDtypeStruct((B, 1), jnp.float32),
  )(final_in, w1t, b1r, w2t, b2r)


def kernel(var_factor_prev_msg, factor_var_prev_msg, factor_prev_marginals,
           var_prev_marginals, factor_potentials, edge_var, batch_factors,
           batch_vars, W1, b1, W2, b2):
  del factor_prev_marginals, var_prev_marginals
  f32 = jnp.float32
  padf = NF_PAD - NF

  def to_planes(x):
    p = x.reshape(NF, 4).T
    return jnp.pad(p, ((0, 0), (0, padf)))

  vfm = to_planes(var_factor_prev_msg)
  fvm = to_planes(factor_var_prev_msg)
  pot = to_planes(factor_potentials)
  idx = edge_var.reshape(NF, 2).T
  dummy = NV + (jnp.arange(padf, dtype=jnp.int32) % 64)
  idx = jnp.concatenate(
      [idx, jnp.broadcast_to(dummy, (2, padf))], axis=1).reshape(2, ROWS, 128)
  bf_col = jnp.pad(batch_factors, (0, padf), constant_values=127
                   ).astype(f32).reshape(NF_PAD, 1)
  bv_col = jnp.pad(batch_vars, (0, NV_PAD - NV), constant_values=127
                   ).astype(f32).reshape(NV_PAD, 1)
  ones_r = jnp.ones((HROWS, 128), f32)

  deg_parts = _sc_degrees(idx, ones_r)
  parts = _sc_scatter(fvm.reshape(4, ROWS, 128), idx)
  vb0, vb1, deg = _tc_combine0(parts.reshape(4, NV_PAD), deg_parts)

  avgs, entvs = [], []
  for _ in range(ITERS):
    g = _sc_gather(vb0, vb1, idx).reshape(4, NF_PAD)
    vfm, fvm, compf = _tc_dense(g, vfm, fvm, pot, bf_col)
    parts = _sc_scatter(fvm.reshape(4, ROWS, 128), idx)
    vb0, vb1, compv = _tc_combine(parts.reshape(4, NV_PAD), deg, bv_col)
    avgs.append(compf[0:4, 0:B].T)
    entvs.append(compv[0:2, 0:B].T)

  final_in = jnp.concatenate(
      [jnp.concatenate([a, -a, e], axis=1) for a, e in zip(avgs, entvs)],
      axis=1)
  return _tc_mlp(final_in, W1.T, b1.reshape(1, -1), W2.T, b2.reshape(1, -1))


# speedup vs baseline: 11.7706x; 11.7706x over previous
import functools

import jax
import jax.numpy as jnp
from jax import lax
from jax.experimental import pallas as pl
from jax.experimental.pallas import tpu as pltpu
from jax.experimental.pallas import tpu_sc as plsc

NV = 100000
NF = 800000
E = 1600000
B = 16
ITERS = 5
DAMP = 0.5

NV_PAD = 102400
ROWS = 6272
NF_PAD = ROWS * 128
NT = 32
ROWS_T = ROWS // NT
HROWS = ROWS_T // 2
VSLICE = NV_PAD // 16
HALFE = HROWS * 128

BLK = 6272
GRID_F = NF_PAD // BLK
BLKV = 5120
GRID_V = NV_PAD // BLKV


@functools.lru_cache(maxsize=None)
def _mesh():
  return plsc.VectorSubcoreMesh(core_axis_name="c", subcore_axis_name="s",
                                num_cores=2, num_subcores=16)


def _zero_shared(zbuf, shared_refs, s):
  def body(i, _):
    zbuf[pl.ds(pl.multiple_of(i * 16, 16), 16)] = jnp.zeros((16,), jnp.float32)
    return 0
  lax.fori_loop(0, VSLICE // 16, body, 0)
  off = pl.multiple_of(s * VSLICE, 8)
  for ref in shared_refs:
    pltpu.sync_copy(zbuf, ref.at[pl.ds(off, VSLICE)])


@functools.lru_cache(maxsize=None)
def _sc_scatter_kernel():
  return functools.partial(
      pl.kernel,
      out_type=jax.ShapeDtypeStruct((2, 2, NV_PAD), jnp.float32),
      mesh=_mesh(),
      scratch_types=[
          pltpu.VMEM((HALFE,), jnp.int32),
          pltpu.VMEM((HALFE,), jnp.int32),
          pltpu.VMEM((HALFE,), jnp.float32),
          pltpu.VMEM((HALFE,), jnp.float32),
          pltpu.VMEM((HALFE,), jnp.float32),
          pltpu.VMEM((HALFE,), jnp.float32),
          pltpu.VMEM((VSLICE,), jnp.float32),
          pltpu.MemorySpace.VMEM_SHARED((NV_PAD,), jnp.float32),
          pltpu.MemorySpace.VMEM_SHARED((NV_PAD,), jnp.float32),
          pltpu.SemaphoreType.DMA,
      ],
  )(_sc_scatter_body)


def _sc_scatter_body(fvm_hbm, idx_hbm, out_hbm, i0, i1, v00, v01, v10, v11,
                     zbuf, t0, t1, sem):
  c = lax.axis_index("c")
  s = lax.axis_index("s")
  wid = c * 16 + s
  _zero_shared(zbuf, (t0, t1), s)
  plsc.subcore_barrier()
  ebase = wid * (2 * HALFE)
  for h in range(2):
    e0 = pl.multiple_of(ebase + h * HALFE, HALFE)
    loads = [
        pltpu.async_copy(idx_hbm.at[0, pl.ds(e0, HALFE)], i0, sem),
        pltpu.async_copy(idx_hbm.at[1, pl.ds(e0, HALFE)], i1, sem),
        pltpu.async_copy(fvm_hbm.at[0, pl.ds(e0, HALFE)], v00, sem),
        pltpu.async_copy(fvm_hbm.at[1, pl.ds(e0, HALFE)], v01, sem),
        pltpu.async_copy(fvm_hbm.at[2, pl.ds(e0, HALFE)], v10, sem),
        pltpu.async_copy(fvm_hbm.at[3, pl.ds(e0, HALFE)], v11, sem),
    ]
    for d in loads:
      d.wait()
    scats = [
        pltpu.async_copy(v00, t0.at[i0], sem, add=True),
        pltpu.async_copy(v01, t1.at[i0], sem, add=True),
        pltpu.async_copy(v10, t0.at[i1], sem, add=True),
        pltpu.async_copy(v11, t1.at[i1], sem, add=True),
    ]
    for d in scats:
      d.wait()
  plsc.subcore_barrier()
  off = pl.multiple_of(s * VSLICE, 8)
  pltpu.sync_copy(t0.at[pl.ds(off, VSLICE)],
                  out_hbm.at[c, 0, pl.ds(off, VSLICE)])
  pltpu.sync_copy(t1.at[pl.ds(off, VSLICE)],
                  out_hbm.at[c, 1, pl.ds(off, VSLICE)])


@functools.lru_cache(maxsize=None)
def _sc_degrees_kernel():
  return functools.partial(
      pl.kernel,
      out_type=jax.ShapeDtypeStruct((2, NV_PAD), jnp.float32),
      mesh=_mesh(),
      scratch_types=[
          pltpu.VMEM((HALFE,), jnp.int32),
          pltpu.VMEM((HALFE,), jnp.int32),
          pltpu.VMEM((HALFE,), jnp.float32),
          pltpu.VMEM((VSLICE,), jnp.float32),
          pltpu.MemorySpace.VMEM_SHARED((NV_PAD,), jnp.float32),
          pltpu.SemaphoreType.DMA,
      ],
  )(_sc_degrees_body)


def _sc_degrees_body(idx_hbm, ones_hbm, out_hbm, i0, i1, vones, zbuf, t0, sem):
  c = lax.axis_index("c")
  s = lax.axis_index("s")
  wid = c * 16 + s
  _zero_shared(zbuf, (t0,), s)
  pltpu.sync_copy(ones_hbm, vones)
  plsc.subcore_barrier()
  ebase = wid * (2 * HALFE)
  for h in range(2):
    e0 = pl.multiple_of(ebase + h * HALFE, HALFE)
    loads = [
        pltpu.async_copy(idx_hbm.at[0, pl.ds(e0, HALFE)], i0, sem),
        pltpu.async_copy(idx_hbm.at[1, pl.ds(e0, HALFE)], i1, sem),
    ]
    for d in loads:
      d.wait()
    scats = [
        pltpu.async_copy(vones, t0.at[i0], sem, add=True),
        pltpu.async_copy(vones, t0.at[i1], sem, add=True),
    ]
    for d in scats:
      d.wait()
  plsc.subcore_barrier()
  off = pl.multiple_of(s * VSLICE, 8)
  pltpu.sync_copy(t0.at[pl.ds(off, VSLICE)], out_hbm.at[c, pl.ds(off, VSLICE)])


@functools.lru_cache(maxsize=None)
def _sc_gather_kernel():
  return functools.partial(
      pl.kernel,
      out_type=jax.ShapeDtypeStruct((4, NF_PAD), jnp.float32),
      mesh=_mesh(),
      scratch_types=[
          pltpu.VMEM((HALFE,), jnp.int32),
          pltpu.VMEM((HALFE,), jnp.int32),
          pltpu.VMEM((HALFE,), jnp.float32),
          pltpu.VMEM((HALFE,), jnp.float32),
          pltpu.VMEM((HALFE,), jnp.float32),
          pltpu.VMEM((HALFE,), jnp.float32),
          pltpu.MemorySpace.VMEM_SHARED((NV_PAD,), jnp.float32),
          pltpu.MemorySpace.VMEM_SHARED((NV_PAD,), jnp.float32),
          pltpu.SemaphoreType.DMA,
      ],
  )(_sc_gather_body)


def _sc_gather_body(vb0_hbm, vb1_hbm, idx_hbm, out_hbm, i0, i1, g00, g01, g10,
                    g11, t0, t1, sem):
  c = lax.axis_index("c")
  s = lax.axis_index("s")
  wid = c * 16 + s
  off = pl.multiple_of(s * VSLICE, 8)
  pltpu.sync_copy(vb0_hbm.at[pl.ds(off, VSLICE)], t0.at[pl.ds(off, VSLICE)])
  pltpu.sync_copy(vb1_hbm.at[pl.ds(off, VSLICE)], t1.at[pl.ds(off, VSLICE)])
  plsc.subcore_barrier()
  ebase = wid * (2 * HALFE)
  for h in range(2):
    e0 = pl.multiple_of(ebase + h * HALFE, HALFE)
    loads = [
        pltpu.async_copy(idx_hbm.at[0, pl.ds(e0, HALFE)], i0, sem),
        pltpu.async_copy(idx_hbm.at[1, pl.ds(e0, HALFE)], i1, sem),
    ]
    for d in loads:
      d.wait()
    gats = [
        pltpu.async_copy(t0.at[i0], g00, sem),
        pltpu.async_copy(t1.at[i0], g01, sem),
        pltpu.async_copy(t0.at[i1], g10, sem),
        pltpu.async_copy(t1.at[i1], g11, sem),
    ]
    for d in gats:
      d.wait()
    stores = [
        pltpu.async_copy(g00, out_hbm.at[0, pl.ds(e0, HALFE)], sem),
        pltpu.async_copy(g01, out_hbm.at[1, pl.ds(e0, HALFE)], sem),
        pltpu.async_copy(g10, out_hbm.at[2, pl.ds(e0, HALFE)], sem),
        pltpu.async_copy(g11, out_hbm.at[3, pl.ds(e0, HALFE)], sem),
    ]
    for d in stores:
      d.wait()


def _lse2(a, b):
  m = jnp.maximum(a, b)
  return m + jnp.log1p(jnp.exp(-jnp.abs(a - b)))


def _seg16(fe, bids_col):
  k, n = fe.shape
  fe8 = jnp.concatenate([fe, jnp.zeros((8 - k, n), jnp.float32)], axis=0)
  cols = lax.broadcasted_iota(jnp.int32, (n, 128), 1).astype(jnp.float32)
  oh = (bids_col == cols).astype(jnp.float32)
  return lax.dot_general(fe8, oh, (((1,), (0,)), ((), ())),
                         precision=lax.Precision.HIGHEST,
                         preferred_element_type=jnp.float32)


def _dense_body(g_ref, vf_ref, fv_ref, po_ref, bf_ref, nvf_ref, nfv_ref,
                comp_ref):
  G = g_ref[...]
  V = vf_ref[...]
  F = fv_ref[...]
  P = po_ref[...]

  def r(x, p):
    return x[p:p + 1, :]

  nv = G - F
  l0 = _lse2(r(nv, 0), r(nv, 1))
  l1 = _lse2(r(nv, 2), r(nv, 3))
  lse_rows = jnp.concatenate([l0, l0, l1, l1], axis=0)
  Vn = DAMP * V + (1.0 - DAMP) * (nv - lse_rows)
  nvf_ref[...] = Vn
  m00 = _lse2(r(P, 0) + r(Vn, 2), r(P, 1) + r(Vn, 3))
  m01 = _lse2(r(P, 2) + r(Vn, 2), r(P, 3) + r(Vn, 3))
  m10 = _lse2(r(P, 0) + r(Vn, 0), r(P, 2) + r(Vn, 1))
  m11 = _lse2(r(P, 1) + r(Vn, 0), r(P, 3) + r(Vn, 1))
  z0 = _lse2(m00, m01)
  z1 = _lse2(m10, m11)
  nf = jnp.concatenate([m00 - z0, m01 - z0, m10 - z1, m11 - z1], axis=0)
  Fn = DAMP * F + (1.0 - DAMP) * nf
  nfv_ref[...] = Fn
  fb0 = r(P, 0) + r(Vn, 0) + r(Vn, 2)
  fb1 = r(P, 1) + r(Vn, 0) + r(Vn, 3)
  fb2 = r(P, 2) + r(Vn, 1) + r(Vn, 2)
  fb3 = r(P, 3) + r(Vn, 1) + r(Vn, 3)
  z = _lse2(_lse2(fb0, fb1), _lse2(fb2, fb3))
  FM = jnp.concatenate([fb0 - z, fb1 - z, fb2 - z, fb3 - z], axis=0)
  fe = jnp.exp(FM) * jnp.where(FM == -jnp.inf, 0.0, FM)
  part = _seg16(fe, bf_ref[...])

  @pl.when(pl.program_id(0) == 0)
  def _():
    comp_ref[...] = jnp.zeros_like(comp_ref)

  comp_ref[...] += part


def _tc_dense(g, vfm, fvm, pot, bf_col):
  return pl.pallas_call(
      _dense_body,
      grid=(GRID_F,),
      in_specs=[
          pl.BlockSpec((4, BLK), lambda i: (0, i)),
          pl.BlockSpec((4, BLK), lambda i: (0, i)),
          pl.BlockSpec((4, BLK), lambda i: (0, i)),
          pl.BlockSpec((4, BLK), lambda i: (0, i)),
          pl.BlockSpec((BLK, 1), lambda i: (i, 0)),
      ],
      out_specs=[
          pl.BlockSpec((4, BLK), lambda i: (0, i)),
          pl.BlockSpec((4, BLK), lambda i: (0, i)),
          pl.BlockSpec((8, 128), lambda i: (0, 0)),
      ],
      out_shape=[
          jax.ShapeDtypeStruct((4, NF_PAD), jnp.float32),
          jax.ShapeDtypeStruct((4, NF_PAD), jnp.float32),
          jax.ShapeDtypeStruct((8, 128), jnp.float32),
      ],
  )(g, vfm, fvm, pot, bf_col)


def _combine0_body(p_ref, dp_ref, vb0_ref, vb1_ref, deg_ref):
  Pp = p_ref[...]
  vb0_ref[...] = Pp[0, :] + Pp[2, :]
  vb1_ref[...] = Pp[1, :] + Pp[3, :]
  Dp = dp_ref[...]
  deg_ref[...] = Dp[0, :] + Dp[1, :]


def _tc_combine0(parts, deg_parts):
  return pl.pallas_call(
      _combine0_body,
      grid=(GRID_V,),
      in_specs=[
          pl.BlockSpec((4, BLKV), lambda i: (0, i)),
          pl.BlockSpec((2, BLKV), lambda i: (0, i)),
      ],
      out_specs=[
          pl.BlockSpec((BLKV,), lambda i: (i,)),
          pl.BlockSpec((BLKV,), lambda i: (i,)),
          pl.BlockSpec((BLKV,), lambda i: (i,)),
      ],
      out_shape=[
          jax.ShapeDtypeStruct((NV_PAD,), jnp.float32),
          jax.ShapeDtypeStruct((NV_PAD,), jnp.float32),
          jax.ShapeDtypeStruct((NV_PAD,), jnp.float32),
      ],
  )(parts, deg_parts)


def _combine_body(p_ref, deg_ref, bv_ref, vb0_ref, vb1_ref, comp_ref):
  Pp = p_ref[...]
  vb0 = Pp[0:1, :] + Pp[2:3, :]
  vb1 = Pp[1:2, :] + Pp[3:4, :]
  vb0_ref[...] = vb0[0, :]
  vb1_ref[...] = vb1[0, :]
  z = _lse2(vb0, vb1)
  vm0 = vb0 - z
  vm1 = vb1 - z
  w = deg_ref[...].reshape(1, -1) - 1.0
  fe0 = jnp.exp(vm0) * jnp.where(vm0 == -jnp.inf, 0.0, vm0) * w
  fe1 = jnp.exp(vm1) * jnp.where(vm1 == -jnp.inf, 0.0, vm1) * w
  part = _seg16(jnp.concatenate([fe0, fe1], axis=0), bv_ref[...])

  @pl.when(pl.program_id(0) == 0)
  def _():
    comp_ref[...] = jnp.zeros_like(comp_ref)

  comp_ref[...] += part


def _tc_combine(parts, deg, bv_col):
  return pl.pallas_call(
      _combine_body,
      grid=(GRID_V,),
      in_specs=[
          pl.BlockSpec((4, BLKV), lambda i: (0, i)),
          pl.BlockSpec((BLKV,), lambda i: (i,)),
          pl.BlockSpec((BLKV, 1), lambda i: (i, 0)),
      ],
      out_specs=[
          pl.BlockSpec((BLKV,), lambda i: (i,)),
          pl.BlockSpec((BLKV,), lambda i: (i,)),
          pl.BlockSpec((8, 128), lambda i: (0, 0)),
      ],
      out_shape=[
          jax.ShapeDtypeStruct((NV_PAD,), jnp.float32),
          jax.ShapeDtypeStruct((NV_PAD,), jnp.float32),
          jax.ShapeDtypeStruct((8, 128), jnp.float32),
      ],
  )(parts, deg, bv_col)


def _mlp_body(fi_ref, w1t_ref, b1_ref, w2t_ref, b2_ref, out_ref):
  h = lax.dot_general(fi_ref[...], w1t_ref[...], (((1,), (0,)), ((), ())),
                      precision=lax.Precision.HIGHEST,
                      preferred_element_type=jnp.float32) + b1_ref[...]
  out_ref[...] = lax.dot_general(h, w2t_ref[...], (((1,), (0,)), ((), ())),
                                 precision=lax.Precision.HIGHEST,
                                 preferred_element_type=jnp.float32
                                 ) + b2_ref[...]


def _tc_mlp(final_in, w1t, b1r, w2t, b2r):
  return pl.pallas_call(
      _mlp_body,
      out_shape=jax.ShapeDtypeStruct((B, 1), jnp.float32),
  )(final_in, w1t, b1r, w2t, b2r)


def kernel(var_factor_prev_msg, factor_var_prev_msg, factor_prev_marginals,
           var_prev_marginals, factor_potentials, edge_var, batch_factors,
           batch_vars, W1, b1, W2, b2):
  del factor_prev_marginals, var_prev_marginals
  f32 = jnp.float32
  padf = NF_PAD - NF

  def to_planes(x):
    p = x.reshape(NF, 4).T
    return jnp.pad(p, ((0, 0), (0, padf)))

  vfm = to_planes(var_factor_prev_msg)
  fvm = to_planes(factor_var_prev_msg)
  pot = to_planes(factor_potentials)
  idx = edge_var.reshape(NF, 2).T
  dummy = NV + (jnp.arange(padf, dtype=jnp.int32) % 64)
  idx = jnp.concatenate([idx, jnp.broadcast_to(dummy, (2, padf))], axis=1)
  bf_col = jnp.pad(batch_factors, (0, padf), constant_values=127
                   ).astype(f32).reshape(NF_PAD, 1)
  bv_col = jnp.pad(batch_vars, (0, NV_PAD - NV), constant_values=127
                   ).astype(f32).reshape(NV_PAD, 1)
  ones_r = jnp.ones((HALFE,), f32)

  deg_parts = _sc_degrees_kernel()(idx, ones_r)
  parts = _sc_scatter_kernel()(fvm, idx)
  vb0, vb1, deg = _tc_combine0(parts.reshape(4, NV_PAD), deg_parts)

  avgs, entvs = [], []
  for _ in range(ITERS):
    g = _sc_gather_kernel()(vb0, vb1, idx)
    vfm, fvm, compf = _tc_dense(g, vfm, fvm, pot, bf_col)
    parts = _sc_scatter_kernel()(fvm, idx)
    vb0, vb1, compv = _tc_combine(parts.reshape(4, NV_PAD), deg, bv_col)
    avgs.append(compf[0:4, 0:B].T)
    entvs.append(compv[0:2, 0:B].T)

  final_in = jnp.concatenate(
      [jnp.concatenate([a, -a, e], axis=1) for a, e in zip(avgs, entvs)],
      axis=1)
  return _tc_mlp(final_in, W1.T, b1.reshape(1, -1), W2.T, b2.reshape(1, -1))
